# baseline (speedup 1.0000x reference)
import functools

import jax
import jax.numpy as jnp
from jax import lax
from jax.experimental import pallas as pl
from jax.experimental.pallas import tpu as pltpu
from jax.experimental.pallas import tpu_sc as plsc

N = 10000
E = 320000
NODE_DIM = 128
EDGE_DIM = 16
D = 128
N_LAYER = 3

NH = 2
E2 = E // NH

NC = 2
NS = 16
NW = NC * NS
EPW = E2 // NW
C = 40
NCHUNK = EPW // C
SN = 624
SREM = N - NS * SN

BN = 2000
BE = 2000

_mesh = plsc.VectorSubcoreMesh(
    core_axis_name="c", subcore_axis_name="s", num_cores=NC, num_subcores=NS)


def _dot(a, b):
    return jnp.dot(a, b, preferred_element_type=jnp.float32)


def _edge_enc_body(x_ref, w_ref, b_ref, o_ref):
    o_ref[...] = _dot(x_ref[...], w_ref[...]) + b_ref[...]


def _make_edge_enc(off):
    return pl.pallas_call(
        _edge_enc_body,
        grid=(E2 // BE,),
        in_specs=[
            pl.BlockSpec((BE, EDGE_DIM), lambda i, off=off: (i + off, 0)),
            pl.BlockSpec((EDGE_DIM, D), lambda i: (0, 0)),
            pl.BlockSpec((1, D), lambda i: (0, 0)),
        ],
        out_specs=pl.BlockSpec((BE, D), lambda i: (i, 0)),
        out_shape=jax.ShapeDtypeStruct((E2, D), jnp.float32),
    )


_edge_enc = [_make_edge_enc(0), _make_edge_enc(E2 // BE)]


def _node_enc_body(x_ref, w_ref, b_ref, ws_ref, bs_ref, wd_ref,
                   u_ref, p_ref, q_ref):
    u = _dot(x_ref[...], w_ref[...]) + b_ref[...]
    u_ref[...] = u
    p_ref[...] = _dot(u, ws_ref[...]) + bs_ref[...]
    q_ref[...] = _dot(u, wd_ref[...])


_node_enc = pl.pallas_call(
    _node_enc_body,
    grid=(N // BN,),
    in_specs=[
        pl.BlockSpec((BN, NODE_DIM), lambda i: (i, 0)),
        pl.BlockSpec((NODE_DIM, D), lambda i: (0, 0)),
        pl.BlockSpec((1, D), lambda i: (0, 0)),
        pl.BlockSpec((D, D), lambda i: (0, 0)),
        pl.BlockSpec((1, D), lambda i: (0, 0)),
        pl.BlockSpec((D, D), lambda i: (0, 0)),
    ],
    out_specs=[pl.BlockSpec((BN, D), lambda i: (i, 0))] * 3,
    out_shape=[jax.ShapeDtypeStruct((N, D), jnp.float32)] * 3,
)


def _edge_upd_body(u_ref, gs_ref, gd_ref, w_ref, o_ref):
    u = u_ref[...]
    pre = _dot(u, w_ref[...]) + gs_ref[...] + gd_ref[...]
    o_ref[...] = u + jnp.maximum(pre, 0.0)


_edge_upd = pl.pallas_call(
    _edge_upd_body,
    grid=(E2 // BE,),
    in_specs=[
        pl.BlockSpec((BE, D), lambda i: (i, 0)),
        pl.BlockSpec((BE, D), lambda i: (i, 0)),
        pl.BlockSpec((BE, D), lambda i: (i, 0)),
        pl.BlockSpec((D, D), lambda i: (0, 0)),
    ],
    out_specs=pl.BlockSpec((BE, D), lambda i: (i, 0)),
    out_shape=jax.ShapeDtypeStruct((E2, D), jnp.float32),
)


def _node_upd_proj_body(u_ref, a0_ref, a1_ref, a2_ref, a3_ref,
                        w1_ref, w2_ref, b_ref,
                        ws_ref, bs_ref, wd_ref, uo_ref, p_ref, q_ref):
    u = u_ref[...]
    agg = (a0_ref[...] + a1_ref[...]) + (a2_ref[...] + a3_ref[...])
    h = _dot(u, w1_ref[...]) + _dot(agg, w2_ref[...]) + b_ref[...]
    un = u + jnp.maximum(h, 0.0)
    uo_ref[...] = un
    p_ref[...] = _dot(un, ws_ref[...]) + bs_ref[...]
    q_ref[...] = _dot(un, wd_ref[...])


_node_upd_proj = pl.pallas_call(
    _node_upd_proj_body,
    grid=(N // BN,),
    in_specs=[pl.BlockSpec((BN, D), lambda i: (i, 0))] * 5 + [
        pl.BlockSpec((D, D), lambda i: (0, 0)),
        pl.BlockSpec((D, D), lambda i: (0, 0)),
        pl.BlockSpec((1, D), lambda i: (0, 0)),
        pl.BlockSpec((D, D), lambda i: (0, 0)),
        pl.BlockSpec((1, D), lambda i: (0, 0)),
        pl.BlockSpec((D, D), lambda i: (0, 0)),
    ],
    out_specs=[pl.BlockSpec((BN, D), lambda i: (i, 0))] * 3,
    out_shape=[jax.ShapeDtypeStruct((N, D), jnp.float32)] * 3,
)


def _node_upd_body(u_ref, a0_ref, a1_ref, a2_ref, a3_ref,
                   w1_ref, w2_ref, b_ref, uo_ref):
    u = u_ref[...]
    agg = (a0_ref[...] + a1_ref[...]) + (a2_ref[...] + a3_ref[...])
    h = _dot(u, w1_ref[...]) + _dot(agg, w2_ref[...]) + b_ref[...]
    uo_ref[...] = u + jnp.maximum(h, 0.0)


_node_upd = pl.pallas_call(
    _node_upd_body,
    grid=(N // BN,),
    in_specs=[pl.BlockSpec((BN, D), lambda i: (i, 0))] * 5 + [
        pl.BlockSpec((D, D), lambda i: (0, 0)),
        pl.BlockSpec((D, D), lambda i: (0, 0)),
        pl.BlockSpec((1, D), lambda i: (0, 0)),
    ],
    out_specs=pl.BlockSpec((BN, D), lambda i: (i, 0)),
    out_shape=jax.ShapeDtypeStruct((N, D), jnp.float32),
)


@functools.partial(
    pl.kernel,
    out_type=[jax.ShapeDtypeStruct((E2, D), jnp.float32),
              jax.ShapeDtypeStruct((E2, D), jnp.float32)],
    mesh=_mesh,
    scratch_types=[
        pltpu.VMEM((NCHUNK, C), jnp.int32),
        pltpu.VMEM((NCHUNK, C), jnp.int32),
        pltpu.VMEM((C, D), jnp.float32),
        pltpu.VMEM((C, D), jnp.float32),
        pltpu.VMEM((C, D), jnp.float32),
        pltpu.VMEM((C, D), jnp.float32),
        pltpu.SemaphoreType.DMA,
        pltpu.SemaphoreType.DMA,
        pltpu.SemaphoreType.DMA,
    ],
)
def _sc_gather(p_hbm, q_hbm, src_hbm, dst_hbm, gs_hbm, gd_hbm,
               sidx, didx, prow0, qrow0, prow1, qrow1, sem_p, sem_q, sem_w):
    wid = lax.axis_index("s") * NC + lax.axis_index("c")
    pltpu.sync_copy(src_hbm.at[wid], sidx)
    pltpu.sync_copy(dst_hbm.at[wid], didx)
    ebase = wid * EPW

    def start_gather(j, pbuf, qbuf):
        return (pltpu.async_copy(p_hbm.at[sidx.at[j]], pbuf, sem_p),
                pltpu.async_copy(q_hbm.at[didx.at[j]], qbuf, sem_q))

    def start_write(j, pbuf, qbuf):
        return (pltpu.async_copy(pbuf, gs_hbm.at[pl.ds(ebase + j * C, C)], sem_w),
                pltpu.async_copy(qbuf, gd_hbm.at[pl.ds(ebase + j * C, C)], sem_w))

    cp, cq = start_gather(0, prow0, qrow0)
    cp.wait()
    cq.wait()

    def body(i, carry):
        jA = 2 * i + 1
        cp, cq = start_gather(jA, prow1, qrow1)
        wp, wq = start_write(jA - 1, prow0, qrow0)
        cp.wait(); cq.wait(); wp.wait(); wq.wait()
        cp, cq = start_gather(jA + 1, prow0, qrow0)
        wp, wq = start_write(jA, prow1, qrow1)
        cp.wait(); cq.wait(); wp.wait(); wq.wait()
        return carry

    lax.fori_loop(0, (NCHUNK - 1) // 2, body, 0)
    wp, wq = start_write(NCHUNK - 1, prow0, qrow0)
    wp.wait()
    wq.wait()


@functools.partial(
    pl.kernel,
    out_type=jax.ShapeDtypeStruct((NC, N, D), jnp.float32),
    mesh=_mesh,
    scratch_types=[
        pltpu.VMEM((NCHUNK, C), jnp.int32),
        pltpu.VMEM((C, D), jnp.float32),
        pltpu.VMEM((C, D), jnp.float32),
        pltpu.VMEM_SHARED((N, D), jnp.float32),
        pltpu.SemaphoreType.DMA,
        pltpu.SemaphoreType.DMA,
    ],
)
def _sc_scatter(uef_hbm, dst_hbm, zeros_hbm, out_hbm,
                didx, rows0, rows1, acc, sem_l, sem_s):
    cid = lax.axis_index("c")
    sid = lax.axis_index("s")
    wid = sid * NC + cid
    pltpu.sync_copy(zeros_hbm.at[pl.ds(sid * SN, SN)],
                    acc.at[pl.ds(sid * SN, SN)])

    @pl.when(sid == NS - 1)
    def _zero_rem():
        pltpu.sync_copy(zeros_hbm.at[pl.ds(NS * SN, SREM)],
                        acc.at[pl.ds(NS * SN, SREM)])

    plsc.subcore_barrier()
    pltpu.sync_copy(dst_hbm.at[wid], didx)
    ebase = wid * EPW

    def start_load(j, buf):
        return pltpu.async_copy(uef_hbm.at[pl.ds(ebase + j * C, C)], buf, sem_l)

    def start_scat(j, buf):
        return pltpu.async_copy(buf, acc.at[didx.at[j]], sem_s, add=True)

    start_load(0, rows0).wait()

    def body(i, carry):
        jA = 2 * i + 1
        lA = start_load(jA, rows1)
        sP = start_scat(jA - 1, rows0)
        lA.wait(); sP.wait()
        lB = start_load(jA + 1, rows0)
        sA = start_scat(jA, rows1)
        lB.wait(); sA.wait()
        return carry

    lax.fori_loop(0, (NCHUNK - 1) // 2, body, 0)
    start_scat(NCHUNK - 1, rows0).wait()
    plsc.subcore_barrier()
    pltpu.sync_copy(acc.at[pl.ds(sid * SN, SN)],
                    out_hbm.at[cid, pl.ds(sid * SN, SN)])

    @pl.when(sid == NS - 1)
    def _out_rem():
        pltpu.sync_copy(acc.at[pl.ds(NS * SN, SREM)],
                        out_hbm.at[cid, pl.ds(NS * SN, SREM)])


def kernel(nf, ef, edge_index, W_node_enc, b_node_enc, W_edge_enc, b_edge_enc,
           We, be, Wn, bn):
    src4 = edge_index[0].reshape(NH, NW, NCHUNK, C)
    dst4 = edge_index[1].reshape(NH, NW, NCHUNK, C)
    zeros_nd = jnp.zeros((N, D), jnp.float32)

    unf, P, Q = _node_enc(nf, W_node_enc, b_node_enc.reshape(1, D),
                          We[0, D:2 * D], be[0].reshape(1, D), We[0, 2 * D:])
    uef = [_edge_enc[h](ef, W_edge_enc, b_edge_enc.reshape(1, D))
           for h in range(NH)]

    for l in range(N_LAYER):
        parts = []
        for h in range(NH):
            gs, gd = _sc_gather(P, Q, src4[h], dst4[h])
            uef[h] = _edge_upd(uef[h], gs, gd, We[l, :D])
            parts.append(_sc_scatter(uef[h], dst4[h], zeros_nd))
        aggs = (parts[0][0], parts[0][1], parts[1][0], parts[1][1])
        if l + 1 < N_LAYER:
            unf, P, Q = _node_upd_proj(
                unf, *aggs,
                Wn[l, :D], Wn[l, D:], bn[l].reshape(1, D),
                We[l + 1, D:2 * D], be[l + 1].reshape(1, D), We[l + 1, 2 * D:])
        else:
            unf = _node_upd(unf, *aggs,
                            Wn[l, :D], Wn[l, D:], bn[l].reshape(1, D))
    return unf, jnp.concatenate(uef, axis=0)


# speedup vs baseline: 1.1718x; 1.1718x over previous
import functools

import jax
import jax.numpy as jnp
from jax import lax
from jax.experimental import pallas as pl
from jax.experimental.pallas import tpu as pltpu
from jax.experimental.pallas import tpu_sc as plsc

N = 10000
E = 320000
NODE_DIM = 128
EDGE_DIM = 16
D = 128
N_LAYER = 3

NH = 2
E2 = E // NH

NC = 2
NS = 16
NW = NC * NS
EPW = E2 // NW
C = 128
NFULL = EPW // C
REM = EPW - NFULL * C
SN = 624
SREM = N - NS * SN

BN = 2000
BE = 2000

_mesh = plsc.VectorSubcoreMesh(
    core_axis_name="c", subcore_axis_name="s", num_cores=NC, num_subcores=NS)


def _dot(a, b):
    return jnp.dot(a, b, preferred_element_type=jnp.float32)


def _edge_enc_body(x_ref, w_ref, b_ref, o_ref):
    o_ref[...] = _dot(x_ref[...], w_ref[...]) + b_ref[...]


def _make_edge_enc(off):
    return pl.pallas_call(
        _edge_enc_body,
        grid=(E2 // BE,),
        in_specs=[
            pl.BlockSpec((BE, EDGE_DIM), lambda i, off=off: (i + off, 0)),
            pl.BlockSpec((EDGE_DIM, D), lambda i: (0, 0)),
            pl.BlockSpec((1, D), lambda i: (0, 0)),
        ],
        out_specs=pl.BlockSpec((BE, D), lambda i: (i, 0)),
        out_shape=jax.ShapeDtypeStruct((E2, D), jnp.float32),
    )


_edge_enc = [_make_edge_enc(0), _make_edge_enc(E2 // BE)]


def _node_enc_body(x_ref, w_ref, b_ref, ws_ref, bs_ref, wd_ref,
                   u_ref, p_ref, q_ref):
    u = _dot(x_ref[...], w_ref[...]) + b_ref[...]
    u_ref[...] = u
    p_ref[...] = _dot(u, ws_ref[...]) + bs_ref[...]
    q_ref[...] = _dot(u, wd_ref[...])


_node_enc = pl.pallas_call(
    _node_enc_body,
    grid=(N // BN,),
    in_specs=[
        pl.BlockSpec((BN, NODE_DIM), lambda i: (i, 0)),
        pl.BlockSpec((NODE_DIM, D), lambda i: (0, 0)),
        pl.BlockSpec((1, D), lambda i: (0, 0)),
        pl.BlockSpec((D, D), lambda i: (0, 0)),
        pl.BlockSpec((1, D), lambda i: (0, 0)),
        pl.BlockSpec((D, D), lambda i: (0, 0)),
    ],
    out_specs=[pl.BlockSpec((BN, D), lambda i: (i, 0))] * 3,
    out_shape=[jax.ShapeDtypeStruct((N, D), jnp.float32)] * 3,
)


def _edge_upd_body(u_ref, gs_ref, gd_ref, w_ref, o_ref):
    u = u_ref[...]
    pre = _dot(u, w_ref[...]) + gs_ref[...] + gd_ref[...]
    o_ref[...] = u + jnp.maximum(pre, 0.0)


_edge_upd = pl.pallas_call(
    _edge_upd_body,
    grid=(E2 // BE,),
    in_specs=[
        pl.BlockSpec((BE, D), lambda i: (i, 0)),
        pl.BlockSpec((BE, D), lambda i: (i, 0)),
        pl.BlockSpec((BE, D), lambda i: (i, 0)),
        pl.BlockSpec((D, D), lambda i: (0, 0)),
    ],
    out_specs=pl.BlockSpec((BE, D), lambda i: (i, 0)),
    out_shape=jax.ShapeDtypeStruct((E2, D), jnp.float32),
)


def _node_upd_proj_body(u_ref, a0_ref, a1_ref, a2_ref, a3_ref,
                        w1_ref, w2_ref, b_ref,
                        ws_ref, bs_ref, wd_ref, uo_ref, p_ref, q_ref):
    u = u_ref[...]
    agg = (a0_ref[...] + a1_ref[...]) + (a2_ref[...] + a3_ref[...])
    h = _dot(u, w1_ref[...]) + _dot(agg, w2_ref[...]) + b_ref[...]
    un = u + jnp.maximum(h, 0.0)
    uo_ref[...] = un
    p_ref[...] = _dot(un, ws_ref[...]) + bs_ref[...]
    q_ref[...] = _dot(un, wd_ref[...])


_node_upd_proj = pl.pallas_call(
    _node_upd_proj_body,
    grid=(N // BN,),
    in_specs=[pl.BlockSpec((BN, D), lambda i: (i, 0))] * 5 + [
        pl.BlockSpec((D, D), lambda i: (0, 0)),
        pl.BlockSpec((D, D), lambda i: (0, 0)),
        pl.BlockSpec((1, D), lambda i: (0, 0)),
        pl.BlockSpec((D, D), lambda i: (0, 0)),
        pl.BlockSpec((1, D), lambda i: (0, 0)),
        pl.BlockSpec((D, D), lambda i: (0, 0)),
    ],
    out_specs=[pl.BlockSpec((BN, D), lambda i: (i, 0))] * 3,
    out_shape=[jax.ShapeDtypeStruct((N, D), jnp.float32)] * 3,
)


def _node_upd_body(u_ref, a0_ref, a1_ref, a2_ref, a3_ref,
                   w1_ref, w2_ref, b_ref, uo_ref):
    u = u_ref[...]
    agg = (a0_ref[...] + a1_ref[...]) + (a2_ref[...] + a3_ref[...])
    h = _dot(u, w1_ref[...]) + _dot(agg, w2_ref[...]) + b_ref[...]
    uo_ref[...] = u + jnp.maximum(h, 0.0)


_node_upd = pl.pallas_call(
    _node_upd_body,
    grid=(N // BN,),
    in_specs=[pl.BlockSpec((BN, D), lambda i: (i, 0))] * 5 + [
        pl.BlockSpec((D, D), lambda i: (0, 0)),
        pl.BlockSpec((D, D), lambda i: (0, 0)),
        pl.BlockSpec((1, D), lambda i: (0, 0)),
    ],
    out_specs=pl.BlockSpec((BN, D), lambda i: (i, 0)),
    out_shape=jax.ShapeDtypeStruct((N, D), jnp.float32),
)


@functools.partial(
    pl.kernel,
    out_type=[jax.ShapeDtypeStruct((E2, D), jnp.float32),
              jax.ShapeDtypeStruct((E2, D), jnp.float32)],
    mesh=_mesh,
    scratch_types=[
        pltpu.VMEM((NFULL, C), jnp.int32),
        pltpu.VMEM((NFULL, C), jnp.int32),
        pltpu.VMEM((1, REM), jnp.int32),
        pltpu.VMEM((1, REM), jnp.int32),
        pltpu.VMEM((C, D), jnp.float32),
        pltpu.VMEM((C, D), jnp.float32),
        pltpu.VMEM((C, D), jnp.float32),
        pltpu.VMEM((C, D), jnp.float32),
        pltpu.SemaphoreType.DMA,
        pltpu.SemaphoreType.DMA,
        pltpu.SemaphoreType.DMA,
    ],
)
def _sc_gather(p_hbm, q_hbm, src_hbm, dst_hbm, srcr_hbm, dstr_hbm,
               gs_hbm, gd_hbm,
               sidx, didx, sidxr, didxr, prow0, qrow0, prow1, qrow1,
               sem_p, sem_q, sem_w):
    wid = lax.axis_index("s") * NC + lax.axis_index("c")
    pltpu.sync_copy(src_hbm.at[wid], sidx)
    pltpu.sync_copy(dst_hbm.at[wid], didx)
    pltpu.sync_copy(srcr_hbm.at[wid], sidxr)
    pltpu.sync_copy(dstr_hbm.at[wid], didxr)
    ebase = wid * EPW

    def start_gather(j, pbuf, qbuf):
        return (pltpu.async_copy(p_hbm.at[sidx.at[j]], pbuf, sem_p),
                pltpu.async_copy(q_hbm.at[didx.at[j]], qbuf, sem_q))

    def start_write(j, pbuf, qbuf):
        return (pltpu.async_copy(pbuf, gs_hbm.at[pl.ds(ebase + j * C, C)], sem_w),
                pltpu.async_copy(qbuf, gd_hbm.at[pl.ds(ebase + j * C, C)], sem_w))

    cp, cq = start_gather(0, prow0, qrow0)
    cp.wait()
    cq.wait()

    def body(i, carry):
        jA = 2 * i + 1
        cp, cq = start_gather(jA, prow1, qrow1)
        wp, wq = start_write(jA - 1, prow0, qrow0)
        cp.wait(); cq.wait(); wp.wait(); wq.wait()
        cp, cq = start_gather(jA + 1, prow0, qrow0)
        wp, wq = start_write(jA, prow1, qrow1)
        cp.wait(); cq.wait(); wp.wait(); wq.wait()
        return carry

    lax.fori_loop(0, (NFULL - 1) // 2, body, 0)
    rp = pltpu.async_copy(p_hbm.at[sidxr.at[0]], prow1.at[pl.ds(0, REM)], sem_p)
    rq = pltpu.async_copy(q_hbm.at[didxr.at[0]], qrow1.at[pl.ds(0, REM)], sem_q)
    wp, wq = start_write(NFULL - 1, prow0, qrow0)
    rp.wait(); rq.wait(); wp.wait(); wq.wait()
    rbase = ebase + NFULL * C
    wp = pltpu.async_copy(prow1.at[pl.ds(0, REM)],
                          gs_hbm.at[pl.ds(rbase, REM)], sem_w)
    wq = pltpu.async_copy(qrow1.at[pl.ds(0, REM)],
                          gd_hbm.at[pl.ds(rbase, REM)], sem_w)
    wp.wait()
    wq.wait()


@functools.partial(
    pl.kernel,
    out_type=jax.ShapeDtypeStruct((NC, N, D), jnp.float32),
    mesh=_mesh,
    scratch_types=[
        pltpu.VMEM((NFULL, C), jnp.int32),
        pltpu.VMEM((1, REM), jnp.int32),
        pltpu.VMEM((C, D), jnp.float32),
        pltpu.VMEM((C, D), jnp.float32),
        pltpu.VMEM_SHARED((N, D), jnp.float32),
        pltpu.SemaphoreType.DMA,
        pltpu.SemaphoreType.DMA,
    ],
)
def _sc_scatter(uef_hbm, dst_hbm, dstr_hbm, zeros_hbm, out_hbm,
                didx, didxr, rows0, rows1, acc, sem_l, sem_s):
    cid = lax.axis_index("c")
    sid = lax.axis_index("s")
    wid = sid * NC + cid
    pltpu.sync_copy(zeros_hbm.at[pl.ds(sid * SN, SN)],
                    acc.at[pl.ds(sid * SN, SN)])

    @pl.when(sid == NS - 1)
    def _zero_rem():
        pltpu.sync_copy(zeros_hbm.at[pl.ds(NS * SN, SREM)],
                        acc.at[pl.ds(NS * SN, SREM)])

    plsc.subcore_barrier()
    pltpu.sync_copy(dst_hbm.at[wid], didx)
    pltpu.sync_copy(dstr_hbm.at[wid], didxr)
    ebase = wid * EPW

    def start_load(j, buf):
        return pltpu.async_copy(uef_hbm.at[pl.ds(ebase + j * C, C)], buf, sem_l)

    def start_scat(j, buf):
        return pltpu.async_copy(buf, acc.at[didx.at[j]], sem_s, add=True)

    start_load(0, rows0).wait()

    def body(i, carry):
        jA = 2 * i + 1
        lA = start_load(jA, rows1)
        sP = start_scat(jA - 1, rows0)
        lA.wait(); sP.wait()
        lB = start_load(jA + 1, rows0)
        sA = start_scat(jA, rows1)
        lB.wait(); sA.wait()
        return carry

    lax.fori_loop(0, (NFULL - 1) // 2, body, 0)
    rbase = ebase + NFULL * C
    rl = pltpu.async_copy(uef_hbm.at[pl.ds(rbase, REM)],
                          rows1.at[pl.ds(0, REM)], sem_l)
    sP = start_scat(NFULL - 1, rows0)
    rl.wait()
    sP.wait()
    pltpu.async_copy(rows1.at[pl.ds(0, REM)], acc.at[didxr.at[0]],
                     sem_s, add=True).wait()
    plsc.subcore_barrier()
    pltpu.sync_copy(acc.at[pl.ds(sid * SN, SN)],
                    out_hbm.at[cid, pl.ds(sid * SN, SN)])

    @pl.when(sid == NS - 1)
    def _out_rem():
        pltpu.sync_copy(acc.at[pl.ds(NS * SN, SREM)],
                        out_hbm.at[cid, pl.ds(NS * SN, SREM)])


def kernel(nf, ef, edge_index, W_node_enc, b_node_enc, W_edge_enc, b_edge_enc,
           We, be, Wn, bn):
    s2 = edge_index[0].reshape(NH, NW, EPW)
    d2 = edge_index[1].reshape(NH, NW, EPW)
    srcm = s2[..., :NFULL * C].reshape(NH, NW, NFULL, C)
    srcr = s2[..., NFULL * C:].reshape(NH, NW, 1, REM)
    dstm = d2[..., :NFULL * C].reshape(NH, NW, NFULL, C)
    dstr = d2[..., NFULL * C:].reshape(NH, NW, 1, REM)
    zeros_nd = jnp.zeros((N, D), jnp.float32)

    unf, P, Q = _node_enc(nf, W_node_enc, b_node_enc.reshape(1, D),
                          We[0, D:2 * D], be[0].reshape(1, D), We[0, 2 * D:])
    uef = [_edge_enc[h](ef, W_edge_enc, b_edge_enc.reshape(1, D))
           for h in range(NH)]

    for l in range(N_LAYER):
        parts = []
        for h in range(NH):
            gs, gd = _sc_gather(P, Q, srcm[h], dstm[h], srcr[h], dstr[h])
            uef[h] = _edge_upd(uef[h], gs, gd, We[l, :D])
            parts.append(_sc_scatter(uef[h], dstm[h], dstr[h], zeros_nd))
        aggs = (parts[0][0], parts[0][1], parts[1][0], parts[1][1])
        if l + 1 < N_LAYER:
            unf, P, Q = _node_upd_proj(
                unf, *aggs,
                Wn[l, :D], Wn[l, D:], bn[l].reshape(1, D),
                We[l + 1, D:2 * D], be[l + 1].reshape(1, D), We[l + 1, 2 * D:])
        else:
            unf = _node_upd(unf, *aggs,
                            Wn[l, :D], Wn[l, D:], bn[l].reshape(1, D))
    return unf, jnp.concatenate(uef, axis=0)


# speedup vs baseline: 1.1821x; 1.0088x over previous
import functools

import jax
import jax.numpy as jnp
from jax import lax
from jax.experimental import pallas as pl
from jax.experimental.pallas import tpu as pltpu
from jax.experimental.pallas import tpu_sc as plsc

N = 10000
E = 320000
NODE_DIM = 128
EDGE_DIM = 16
D = 128
N_LAYER = 3

NH = 2
E2 = E // NH

NC = 2
NS = 16
NW = NC * NS
EPW = E2 // NW
CB = 256
NBIG = 19
CT = 128
NIDX = NBIG * CB + CT
CBS = 128
NCHS = NIDX // CBS
REM = EPW - NIDX
SN = 624
SREM = N - NS * SN

BN = 2000
BE = 2000

_mesh = plsc.VectorSubcoreMesh(
    core_axis_name="c", subcore_axis_name="s", num_cores=NC, num_subcores=NS)


def _dot(a, b):
    return jnp.dot(a, b, preferred_element_type=jnp.float32)


def _edge_enc_body(x_ref, w_ref, b_ref, o_ref):
    o_ref[...] = _dot(x_ref[...], w_ref[...]) + b_ref[...]


def _make_edge_enc(off):
    return pl.pallas_call(
        _edge_enc_body,
        grid=(E2 // BE,),
        in_specs=[
            pl.BlockSpec((BE, EDGE_DIM), lambda i, off=off: (i + off, 0)),
            pl.BlockSpec((EDGE_DIM, D), lambda i: (0, 0)),
            pl.BlockSpec((1, D), lambda i: (0, 0)),
        ],
        out_specs=pl.BlockSpec((BE, D), lambda i: (i, 0)),
        out_shape=jax.ShapeDtypeStruct((E2, D), jnp.float32),
    )


_edge_enc = [_make_edge_enc(0), _make_edge_enc(E2 // BE)]


def _node_enc_body(x_ref, w_ref, b_ref, ws_ref, bs_ref, wd_ref,
                   u_ref, p_ref, q_ref):
    u = _dot(x_ref[...], w_ref[...]) + b_ref[...]
    u_ref[...] = u
    p_ref[...] = _dot(u, ws_ref[...]) + bs_ref[...]
    q_ref[...] = _dot(u, wd_ref[...])


_node_enc = pl.pallas_call(
    _node_enc_body,
    grid=(N // BN,),
    in_specs=[
        pl.BlockSpec((BN, NODE_DIM), lambda i: (i, 0)),
        pl.BlockSpec((NODE_DIM, D), lambda i: (0, 0)),
        pl.BlockSpec((1, D), lambda i: (0, 0)),
        pl.BlockSpec((D, D), lambda i: (0, 0)),
        pl.BlockSpec((1, D), lambda i: (0, 0)),
        pl.BlockSpec((D, D), lambda i: (0, 0)),
    ],
    out_specs=[pl.BlockSpec((BN, D), lambda i: (i, 0))] * 3,
    out_shape=[jax.ShapeDtypeStruct((N, D), jnp.float32)] * 3,
)


def _edge_upd_body(u_ref, gs_ref, gd_ref, w_ref, o_ref):
    u = u_ref[...]
    pre = _dot(u, w_ref[...]) + gs_ref[...] + gd_ref[...]
    o_ref[...] = u + jnp.maximum(pre, 0.0)


_edge_upd = pl.pallas_call(
    _edge_upd_body,
    grid=(E2 // BE,),
    in_specs=[
        pl.BlockSpec((BE, D), lambda i: (i, 0)),
        pl.BlockSpec((BE, D), lambda i: (i, 0)),
        pl.BlockSpec((BE, D), lambda i: (i, 0)),
        pl.BlockSpec((D, D), lambda i: (0, 0)),
    ],
    out_specs=pl.BlockSpec((BE, D), lambda i: (i, 0)),
    out_shape=jax.ShapeDtypeStruct((E2, D), jnp.float32),
)


def _node_upd_proj_body(u_ref, a0_ref, a1_ref,
                        w1_ref, w2_ref, b_ref,
                        ws_ref, bs_ref, wd_ref, uo_ref, p_ref, q_ref):
    u = u_ref[...]
    agg = a0_ref[...] + a1_ref[...]
    h = _dot(u, w1_ref[...]) + _dot(agg, w2_ref[...]) + b_ref[...]
    un = u + jnp.maximum(h, 0.0)
    uo_ref[...] = un
    p_ref[...] = _dot(un, ws_ref[...]) + bs_ref[...]
    q_ref[...] = _dot(un, wd_ref[...])


_node_upd_proj = pl.pallas_call(
    _node_upd_proj_body,
    grid=(N // BN,),
    in_specs=[pl.BlockSpec((BN, D), lambda i: (i, 0))] * 3 + [
        pl.BlockSpec((D, D), lambda i: (0, 0)),
        pl.BlockSpec((D, D), lambda i: (0, 0)),
        pl.BlockSpec((1, D), lambda i: (0, 0)),
        pl.BlockSpec((D, D), lambda i: (0, 0)),
        pl.BlockSpec((1, D), lambda i: (0, 0)),
        pl.BlockSpec((D, D), lambda i: (0, 0)),
    ],
    out_specs=[pl.BlockSpec((BN, D), lambda i: (i, 0))] * 3,
    out_shape=[jax.ShapeDtypeStruct((N, D), jnp.float32)] * 3,
)


def _node_upd_body(u_ref, a0_ref, a1_ref,
                   w1_ref, w2_ref, b_ref, uo_ref):
    u = u_ref[...]
    agg = a0_ref[...] + a1_ref[...]
    h = _dot(u, w1_ref[...]) + _dot(agg, w2_ref[...]) + b_ref[...]
    uo_ref[...] = u + jnp.maximum(h, 0.0)


_node_upd = pl.pallas_call(
    _node_upd_body,
    grid=(N // BN,),
    in_specs=[pl.BlockSpec((BN, D), lambda i: (i, 0))] * 3 + [
        pl.BlockSpec((D, D), lambda i: (0, 0)),
        pl.BlockSpec((D, D), lambda i: (0, 0)),
        pl.BlockSpec((1, D), lambda i: (0, 0)),
    ],
    out_specs=pl.BlockSpec((BN, D), lambda i: (i, 0)),
    out_shape=jax.ShapeDtypeStruct((N, D), jnp.float32),
)


@functools.partial(
    pl.kernel,
    out_type=[jax.ShapeDtypeStruct((E2, D), jnp.float32),
              jax.ShapeDtypeStruct((E2, D), jnp.float32)],
    mesh=_mesh,
    scratch_types=[
        pltpu.VMEM((NIDX,), jnp.int32),
        pltpu.VMEM((NIDX,), jnp.int32),
        pltpu.VMEM((1, REM), jnp.int32),
        pltpu.VMEM((1, REM), jnp.int32),
        pltpu.VMEM((CB, D), jnp.float32),
        pltpu.VMEM((CB, D), jnp.float32),
        pltpu.SemaphoreType.DMA,
        pltpu.SemaphoreType.DMA,
        pltpu.SemaphoreType.DMA,
    ],
)
def _sc_gather(p_hbm, q_hbm, src_hbm, dst_hbm, srcr_hbm, dstr_hbm,
               gs_hbm, gd_hbm,
               sidx, didx, sidxr, didxr, pbuf, qbuf, sem_p, sem_q, sem_w):
    wid = lax.axis_index("s") * NC + lax.axis_index("c")
    pltpu.sync_copy(src_hbm.at[wid], sidx)
    pltpu.sync_copy(dst_hbm.at[wid], didx)
    pltpu.sync_copy(srcr_hbm.at[wid], sidxr)
    pltpu.sync_copy(dstr_hbm.at[wid], didxr)
    ebase = wid * EPW

    def gat(tbl, idx, off, sz, buf, sem):
        return pltpu.async_copy(tbl.at[idx.at[pl.ds(off, sz)]],
                                buf.at[pl.ds(0, sz)], sem)

    def wr(buf, out, off, sz):
        return pltpu.async_copy(buf.at[pl.ds(0, sz)],
                                out.at[pl.ds(ebase + off, sz)], sem_w)

    gat(p_hbm, sidx, 0, CB, pbuf, sem_p).wait()

    def body(j, carry):
        off = j * CB
        wP = wr(pbuf, gs_hbm, off, CB)
        gQ = gat(q_hbm, didx, off, CB, qbuf, sem_q)
        wP.wait(); gQ.wait()
        wQ = wr(qbuf, gd_hbm, off, CB)
        gP = gat(p_hbm, sidx, off + CB, CB, pbuf, sem_p)
        wQ.wait(); gP.wait()
        return carry

    lax.fori_loop(0, NBIG - 1, body, 0)
    off = (NBIG - 1) * CB
    wP = wr(pbuf, gs_hbm, off, CB)
    gQ = gat(q_hbm, didx, off, CB, qbuf, sem_q)
    wP.wait(); gQ.wait()
    wQ = wr(qbuf, gd_hbm, off, CB)
    gP = gat(p_hbm, sidx, NBIG * CB, CT, pbuf, sem_p)
    wQ.wait(); gP.wait()
    off = NBIG * CB
    wP = wr(pbuf, gs_hbm, off, CT)
    gQ = gat(q_hbm, didx, off, CT, qbuf, sem_q)
    wP.wait(); gQ.wait()
    wQ = wr(qbuf, gd_hbm, off, CT)
    gP = pltpu.async_copy(p_hbm.at[sidxr.at[0]], pbuf.at[pl.ds(0, REM)], sem_p)
    wQ.wait(); gP.wait()
    off = NIDX
    wP = wr(pbuf, gs_hbm, off, REM)
    gQ = pltpu.async_copy(q_hbm.at[didxr.at[0]], qbuf.at[pl.ds(0, REM)], sem_q)
    wP.wait(); gQ.wait()
    wr(qbuf, gd_hbm, off, REM).wait()


@functools.partial(
    pl.kernel,
    out_type=jax.ShapeDtypeStruct((NC, N, D), jnp.float32),
    mesh=_mesh,
    scratch_types=[
        pltpu.VMEM((NIDX,), jnp.int32),
        pltpu.VMEM((1, REM), jnp.int32),
        pltpu.VMEM((CBS, D), jnp.float32),
        pltpu.VMEM((CBS, D), jnp.float32),
        pltpu.VMEM_SHARED((N, D), jnp.float32),
        pltpu.SemaphoreType.DMA,
        pltpu.SemaphoreType.DMA,
    ],
)
def _sc_scatter(uef_hbm, dst_hbm, dstr_hbm, init_hbm, out_hbm,
                didx, didxr, rows0, rows1, acc, sem_l, sem_s):
    cid = lax.axis_index("c")
    sid = lax.axis_index("s")
    wid = sid * NC + cid
    pltpu.sync_copy(init_hbm.at[cid, pl.ds(sid * SN, SN)],
                    acc.at[pl.ds(sid * SN, SN)])

    @pl.when(sid == NS - 1)
    def _zero_rem():
        pltpu.sync_copy(init_hbm.at[cid, pl.ds(NS * SN, SREM)],
                        acc.at[pl.ds(NS * SN, SREM)])

    plsc.subcore_barrier()
    pltpu.sync_copy(dst_hbm.at[wid], didx)
    pltpu.sync_copy(dstr_hbm.at[wid], didxr)
    ebase = wid * EPW

    def start_load(off, sz, buf):
        return pltpu.async_copy(uef_hbm.at[pl.ds(ebase + off, sz)],
                                buf.at[pl.ds(0, sz)], sem_l)

    def start_scat(off, sz, buf):
        return pltpu.async_copy(buf.at[pl.ds(0, sz)],
                                acc.at[didx.at[pl.ds(off, sz)]],
                                sem_s, add=True)

    start_load(0, CBS, rows0).wait()

    def body(i, carry):
        jA = 2 * i + 1
        lA = start_load(jA * CBS, CBS, rows1)
        sP = start_scat((jA - 1) * CBS, CBS, rows0)
        lA.wait(); sP.wait()
        lB = start_load((jA + 1) * CBS, CBS, rows0)
        sA = start_scat(jA * CBS, CBS, rows1)
        lB.wait(); sA.wait()
        return carry

    lax.fori_loop(0, (NCHS - 1) // 2, body, 0)
    lR = pltpu.async_copy(uef_hbm.at[pl.ds(ebase + NIDX, REM)],
                          rows1.at[pl.ds(0, REM)], sem_l)
    sP = start_scat((NCHS - 1) * CBS, CBS, rows0)
    lR.wait(); sP.wait()
    pltpu.async_copy(rows1.at[pl.ds(0, REM)], acc.at[didxr.at[0]],
                     sem_s, add=True).wait()
    plsc.subcore_barrier()
    pltpu.sync_copy(acc.at[pl.ds(sid * SN, SN)],
                    out_hbm.at[cid, pl.ds(sid * SN, SN)])

    @pl.when(sid == NS - 1)
    def _out_rem():
        pltpu.sync_copy(acc.at[pl.ds(NS * SN, SREM)],
                        out_hbm.at[cid, pl.ds(NS * SN, SREM)])


def kernel(nf, ef, edge_index, W_node_enc, b_node_enc, W_edge_enc, b_edge_enc,
           We, be, Wn, bn):
    s2 = edge_index[0].reshape(NH, NW, EPW)
    d2 = edge_index[1].reshape(NH, NW, EPW)
    srcm = s2[..., :NIDX]
    srcr = s2[..., NIDX:].reshape(NH, NW, 1, REM)
    dstm = d2[..., :NIDX]
    dstr = d2[..., NIDX:].reshape(NH, NW, 1, REM)
    zeros_nd = jnp.zeros((NC, N, D), jnp.float32)

    unf, P, Q = _node_enc(nf, W_node_enc, b_node_enc.reshape(1, D),
                          We[0, D:2 * D], be[0].reshape(1, D), We[0, 2 * D:])
    uef = [_edge_enc[h](ef, W_edge_enc, b_edge_enc.reshape(1, D))
           for h in range(NH)]

    for l in range(N_LAYER):
        part = zeros_nd
        for h in range(NH):
            gs, gd = _sc_gather(P, Q, srcm[h], dstm[h], srcr[h], dstr[h])
            uef[h] = _edge_upd(uef[h], gs, gd, We[l, :D])
            part = _sc_scatter(uef[h], dstm[h], dstr[h], part)
        aggs = (part[0], part[1])
        if l + 1 < N_LAYER:
            unf, P, Q = _node_upd_proj(
                unf, *aggs,
                Wn[l, :D], Wn[l, D:], bn[l].reshape(1, D),
                We[l + 1, D:2 * D], be[l + 1].reshape(1, D), We[l + 1, 2 * D:])
        else:
            unf = _node_upd(unf, *aggs,
                            Wn[l, :D], Wn[l, D:], bn[l].reshape(1, D))
    return unf, jnp.concatenate(uef, axis=0)


# speedup vs baseline: 1.3208x; 1.1173x over previous
import functools

import jax
import jax.numpy as jnp
from jax import lax
from jax.experimental import pallas as pl
from jax.experimental.pallas import tpu as pltpu
from jax.experimental.pallas import tpu_sc as plsc

N = 10000
E = 320000
NODE_DIM = 128
EDGE_DIM = 16
D = 128
N_LAYER = 3

NH = 2
E2 = E // NH

NC = 2
NS = 16
NW = NC * NS
EPW = E2 // NW
CB = 256
NBIG = 19
CT = 128
NIDX = NBIG * CB + CT
CBS = 128
NCHS = NIDX // CBS
CG = 128
NCG = NIDX // CG
REM = EPW - NIDX
SN = 624
SREM = N - NS * SN

BN = 2000
BE = 2000

_mesh = plsc.VectorSubcoreMesh(
    core_axis_name="c", subcore_axis_name="s", num_cores=NC, num_subcores=NS)


def _dot(a, b):
    return jnp.dot(a, b, preferred_element_type=jnp.float32)


def _edge_enc_body(x_ref, w_ref, b_ref, o_ref):
    o_ref[...] = _dot(x_ref[...], w_ref[...]) + b_ref[...]


def _make_edge_enc(off):
    return pl.pallas_call(
        _edge_enc_body,
        grid=(E2 // BE,),
        in_specs=[
            pl.BlockSpec((BE, EDGE_DIM), lambda i, off=off: (i + off, 0)),
            pl.BlockSpec((EDGE_DIM, D), lambda i: (0, 0)),
            pl.BlockSpec((1, D), lambda i: (0, 0)),
        ],
        out_specs=pl.BlockSpec((BE, D), lambda i: (i, 0)),
        out_shape=jax.ShapeDtypeStruct((E2, D), jnp.float32),
    )


_edge_enc = [_make_edge_enc(0), _make_edge_enc(E2 // BE)]


def _node_enc_body(x_ref, w_ref, b_ref, ws_ref, bs_ref, wd_ref,
                   u_ref, p_ref, q_ref):
    u = _dot(x_ref[...], w_ref[...]) + b_ref[...]
    u_ref[...] = u
    p_ref[...] = _dot(u, ws_ref[...]) + bs_ref[...]
    q_ref[...] = _dot(u, wd_ref[...])


_node_enc = pl.pallas_call(
    _node_enc_body,
    grid=(N // BN,),
    in_specs=[
        pl.BlockSpec((BN, NODE_DIM), lambda i: (i, 0)),
        pl.BlockSpec((NODE_DIM, D), lambda i: (0, 0)),
        pl.BlockSpec((1, D), lambda i: (0, 0)),
        pl.BlockSpec((D, D), lambda i: (0, 0)),
        pl.BlockSpec((1, D), lambda i: (0, 0)),
        pl.BlockSpec((D, D), lambda i: (0, 0)),
    ],
    out_specs=[pl.BlockSpec((BN, D), lambda i: (i, 0))] * 3,
    out_shape=[jax.ShapeDtypeStruct((N, D), jnp.float32)] * 3,
)


def _edge_upd_body(u_ref, g_ref, w_ref, o_ref):
    u = u_ref[...]
    pre = _dot(u, w_ref[...]) + g_ref[...]
    o_ref[...] = u + jnp.maximum(pre, 0.0)


_edge_upd = pl.pallas_call(
    _edge_upd_body,
    grid=(E2 // BE,),
    in_specs=[
        pl.BlockSpec((BE, D), lambda i: (i, 0)),
        pl.BlockSpec((BE, D), lambda i: (i, 0)),
        pl.BlockSpec((D, D), lambda i: (0, 0)),
    ],
    out_specs=pl.BlockSpec((BE, D), lambda i: (i, 0)),
    out_shape=jax.ShapeDtypeStruct((E2, D), jnp.float32),
)


def _node_upd_proj_body(u_ref, a0_ref, a1_ref,
                        w1_ref, w2_ref, b_ref,
                        ws_ref, bs_ref, wd_ref, uo_ref, p_ref, q_ref):
    u = u_ref[...]
    agg = a0_ref[...] + a1_ref[...]
    h = _dot(u, w1_ref[...]) + _dot(agg, w2_ref[...]) + b_ref[...]
    un = u + jnp.maximum(h, 0.0)
    uo_ref[...] = un
    p_ref[...] = _dot(un, ws_ref[...]) + bs_ref[...]
    q_ref[...] = _dot(un, wd_ref[...])


_node_upd_proj = pl.pallas_call(
    _node_upd_proj_body,
    grid=(N // BN,),
    in_specs=[pl.BlockSpec((BN, D), lambda i: (i, 0))] * 3 + [
        pl.BlockSpec((D, D), lambda i: (0, 0)),
        pl.BlockSpec((D, D), lambda i: (0, 0)),
        pl.BlockSpec((1, D), lambda i: (0, 0)),
        pl.BlockSpec((D, D), lambda i: (0, 0)),
        pl.BlockSpec((1, D), lambda i: (0, 0)),
        pl.BlockSpec((D, D), lambda i: (0, 0)),
    ],
    out_specs=[pl.BlockSpec((BN, D), lambda i: (i, 0))] * 3,
    out_shape=[jax.ShapeDtypeStruct((N, D), jnp.float32)] * 3,
)


def _node_upd_body(u_ref, a0_ref, a1_ref,
                   w1_ref, w2_ref, b_ref, uo_ref):
    u = u_ref[...]
    agg = a0_ref[...] + a1_ref[...]
    h = _dot(u, w1_ref[...]) + _dot(agg, w2_ref[...]) + b_ref[...]
    uo_ref[...] = u + jnp.maximum(h, 0.0)


_node_upd = pl.pallas_call(
    _node_upd_body,
    grid=(N // BN,),
    in_specs=[pl.BlockSpec((BN, D), lambda i: (i, 0))] * 3 + [
        pl.BlockSpec((D, D), lambda i: (0, 0)),
        pl.BlockSpec((D, D), lambda i: (0, 0)),
        pl.BlockSpec((1, D), lambda i: (0, 0)),
    ],
    out_specs=pl.BlockSpec((BN, D), lambda i: (i, 0)),
    out_shape=jax.ShapeDtypeStruct((N, D), jnp.float32),
)


@functools.partial(
    pl.kernel,
    out_type=jax.ShapeDtypeStruct((E2, D), jnp.float32),
    mesh=_mesh,
    scratch_types=[
        pltpu.VMEM((NIDX,), jnp.int32),
        pltpu.VMEM((NIDX,), jnp.int32),
        pltpu.VMEM((1, REM), jnp.int32),
        pltpu.VMEM((1, REM), jnp.int32),
        pltpu.VMEM((CG,), jnp.int32),
        pltpu.VMEM((CG,), jnp.int32),
        pltpu.VMEM((CG, D), jnp.float32),
        pltpu.VMEM((CG, D), jnp.float32),
        pltpu.VMEM((CG, D), jnp.float32),
        pltpu.VMEM((CG, D), jnp.float32),
        pltpu.VMEM_SHARED((NS * CG, D), jnp.float32),
        pltpu.VMEM_SHARED((NS * CG, D), jnp.float32),
        pltpu.SemaphoreType.DMA,
        pltpu.SemaphoreType.DMA,
        pltpu.SemaphoreType.DMA,
        pltpu.SemaphoreType.DMA,
        pltpu.SemaphoreType.DMA,
    ],
)
def _sc_gather(p_hbm, q_hbm, src_hbm, dst_hbm, srcr_hbm, dstr_hbm, iota_hbm,
               g_hbm,
               sidx, didx, sidxr, didxr, iot, myidx, pA, pB, qA, qB, fA, fB,
               sem_p, sem_q, sem_c, sem_a, sem_w):
    cid = lax.axis_index("c")
    sid = lax.axis_index("s")
    wid = sid * NC + cid
    pltpu.sync_copy(src_hbm.at[wid], sidx)
    pltpu.sync_copy(dst_hbm.at[wid], didx)
    pltpu.sync_copy(srcr_hbm.at[wid], sidxr)
    pltpu.sync_copy(dstr_hbm.at[wid], didxr)
    pltpu.sync_copy(iota_hbm, iot)
    base = sid * CG
    for k in range(CG // 16):
        sl = pl.ds(k * 16, 16)
        myidx[sl] = iot[sl] + base
    ebase = wid * EPW

    def gat2(j, pb, qb):
        hp = pltpu.async_copy(p_hbm.at[sidx.at[pl.ds(j * CG, CG)]], pb, sem_p)
        hq = pltpu.async_copy(q_hbm.at[didx.at[pl.ds(j * CG, CG)]], qb, sem_q)
        return hp, hq

    def cp(pb, fb, sz):
        return pltpu.async_copy(pb.at[pl.ds(0, sz)],
                                fb.at[pl.ds(base, sz)], sem_c)

    def addl(fb, qb, sz):
        return pltpu.async_copy(qb.at[pl.ds(0, sz)],
                                fb.at[myidx.at[pl.ds(0, sz)]], sem_a, add=True)

    def wr(fb, off, sz):
        return pltpu.async_copy(fb.at[pl.ds(base, sz)],
                                g_hbm.at[pl.ds(ebase + off, sz)], sem_w)

    def body(i, carry):
        jA = 2 * i
        jB = jA + 1
        hpA, hqA = gat2(jA, pA, qA)
        hpA.wait()
        cA = cp(pA, fA, CG)
        hqA.wait(); cA.wait()
        aA = addl(fA, qA, CG)
        hpB, hqB = gat2(jB, pB, qB)
        aA.wait()
        wA = wr(fA, jA * CG, CG)
        hpB.wait()
        cB = cp(pB, fB, CG)
        hqB.wait(); cB.wait()
        aB = addl(fB, qB, CG)
        wA.wait(); aB.wait()
        wr(fB, jB * CG, CG).wait()
        return carry

    lax.fori_loop(0, NCG // 2, body, 0)
    hp, hq = gat2(NCG - 1, pA, qA)
    hp.wait()
    cA = cp(pA, fA, CG)
    hq.wait(); cA.wait()
    addl(fA, qA, CG).wait()
    wA = wr(fA, (NCG - 1) * CG, CG)
    hp = pltpu.async_copy(p_hbm.at[sidxr.at[0]], pB.at[pl.ds(0, REM)], sem_p)
    hq = pltpu.async_copy(q_hbm.at[didxr.at[0]], qB.at[pl.ds(0, REM)], sem_q)
    hp.wait()
    cB = cp(pB, fB, REM)
    hq.wait(); cB.wait()
    addl(fB, qB, REM).wait()
    wA.wait()
    wr(fB, NIDX, REM).wait()


@functools.partial(
    pl.kernel,
    out_type=jax.ShapeDtypeStruct((NC, N, D), jnp.float32),
    mesh=_mesh,
    scratch_types=[
        pltpu.VMEM((NIDX,), jnp.int32),
        pltpu.VMEM((1, REM), jnp.int32),
        pltpu.VMEM((CBS, D), jnp.float32),
        pltpu.VMEM((CBS, D), jnp.float32),
        pltpu.VMEM_SHARED((N, D), jnp.float32),
        pltpu.SemaphoreType.DMA,
        pltpu.SemaphoreType.DMA,
    ],
)
def _sc_scatter(uef_hbm, dst_hbm, dstr_hbm, init_hbm, out_hbm,
                didx, didxr, rows0, rows1, acc, sem_l, sem_s):
    cid = lax.axis_index("c")
    sid = lax.axis_index("s")
    wid = sid * NC + cid
    pltpu.sync_copy(init_hbm.at[cid, pl.ds(sid * SN, SN)],
                    acc.at[pl.ds(sid * SN, SN)])

    @pl.when(sid == NS - 1)
    def _zero_rem():
        pltpu.sync_copy(init_hbm.at[cid, pl.ds(NS * SN, SREM)],
                        acc.at[pl.ds(NS * SN, SREM)])

    plsc.subcore_barrier()
    pltpu.sync_copy(dst_hbm.at[wid], didx)
    pltpu.sync_copy(dstr_hbm.at[wid], didxr)
    ebase = wid * EPW

    def start_load(off, sz, buf):
        return pltpu.async_copy(uef_hbm.at[pl.ds(ebase + off, sz)],
                                buf.at[pl.ds(0, sz)], sem_l)

    def start_scat(off, sz, buf):
        return pltpu.async_copy(buf.at[pl.ds(0, sz)],
                                acc.at[didx.at[pl.ds(off, sz)]],
                                sem_s, add=True)

    start_load(0, CBS, rows0).wait()

    def body(i, carry):
        jA = 2 * i + 1
        lA = start_load(jA * CBS, CBS, rows1)
        sP = start_scat((jA - 1) * CBS, CBS, rows0)
        lA.wait(); sP.wait()
        lB = start_load((jA + 1) * CBS, CBS, rows0)
        sA = start_scat(jA * CBS, CBS, rows1)
        lB.wait(); sA.wait()
        return carry

    lax.fori_loop(0, (NCHS - 1) // 2, body, 0)
    lR = pltpu.async_copy(uef_hbm.at[pl.ds(ebase + NIDX, REM)],
                          rows1.at[pl.ds(0, REM)], sem_l)
    sP = start_scat((NCHS - 1) * CBS, CBS, rows0)
    lR.wait(); sP.wait()
    pltpu.async_copy(rows1.at[pl.ds(0, REM)], acc.at[didxr.at[0]],
                     sem_s, add=True).wait()
    plsc.subcore_barrier()
    pltpu.sync_copy(acc.at[pl.ds(sid * SN, SN)],
                    out_hbm.at[cid, pl.ds(sid * SN, SN)])

    @pl.when(sid == NS - 1)
    def _out_rem():
        pltpu.sync_copy(acc.at[pl.ds(NS * SN, SREM)],
                        out_hbm.at[cid, pl.ds(NS * SN, SREM)])


def kernel(nf, ef, edge_index, W_node_enc, b_node_enc, W_edge_enc, b_edge_enc,
           We, be, Wn, bn):
    s2 = edge_index[0].reshape(NH, NW, EPW)
    d2 = edge_index[1].reshape(NH, NW, EPW)
    srcm = s2[..., :NIDX]
    srcr = s2[..., NIDX:].reshape(NH, NW, 1, REM)
    dstm = d2[..., :NIDX]
    dstr = d2[..., NIDX:].reshape(NH, NW, 1, REM)
    zeros_nd = jnp.zeros((NC, N, D), jnp.float32)
    iota = jnp.arange(CG, dtype=jnp.int32)

    unf, P, Q = _node_enc(nf, W_node_enc, b_node_enc.reshape(1, D),
                          We[0, D:2 * D], be[0].reshape(1, D), We[0, 2 * D:])
    uef = [_edge_enc[h](ef, W_edge_enc, b_edge_enc.reshape(1, D))
           for h in range(NH)]

    for l in range(N_LAYER):
        part = zeros_nd
        for h in range(NH):
            g = _sc_gather(P, Q, srcm[h], dstm[h], srcr[h], dstr[h], iota)
            uef[h] = _edge_upd(uef[h], g, We[l, :D])
            part = _sc_scatter(uef[h], dstm[h], dstr[h], part)
        aggs = (part[0], part[1])
        if l + 1 < N_LAYER:
            unf, P, Q = _node_upd_proj(
                unf, *aggs,
                Wn[l, :D], Wn[l, D:], bn[l].reshape(1, D),
                We[l + 1, D:2 * D], be[l + 1].reshape(1, D), We[l + 1, 2 * D:])
        else:
            unf = _node_upd(unf, *aggs,
                            Wn[l, :D], Wn[l, D:], bn[l].reshape(1, D))
    return unf, jnp.concatenate(uef, axis=0)


# speedup vs baseline: 1.4179x; 1.0736x over previous
import functools

import jax
import jax.numpy as jnp
from jax import lax
from jax.experimental import pallas as pl
from jax.experimental.pallas import tpu as pltpu
from jax.experimental.pallas import tpu_sc as plsc

N = 10000
E = 320000
NODE_DIM = 128
EDGE_DIM = 16
D = 128
N_LAYER = 3

NH = 2
E2 = E // NH

NC = 2
NS = 16
NW = NC * NS
EPW = E2 // NW
CB = 256
NBIG = 19
CT = 128
NIDX = NBIG * CB + CT
CBS = 128
NCHS = NIDX // CBS
CG = 128
NCG = NIDX // CG
REM = EPW - NIDX
SN = 624
SREM = N - NS * SN

BN = 2000
BE = 2000

_mesh = plsc.VectorSubcoreMesh(
    core_axis_name="c", subcore_axis_name="s", num_cores=NC, num_subcores=NS)


def _dot(a, b):
    return jnp.dot(a, b, preferred_element_type=jnp.float32)


def _edge_upd0_body(x_ref, we_ref, be_ref, g_ref, w_ref, o_ref):
    u = _dot(x_ref[...], we_ref[...]) + be_ref[...]
    pre = _dot(u, w_ref[...]) + g_ref[...]
    o_ref[...] = u + jnp.maximum(pre, 0.0)


def _make_edge_upd0(off):
    return pl.pallas_call(
        _edge_upd0_body,
        grid=(E2 // BE,),
        in_specs=[
            pl.BlockSpec((BE, EDGE_DIM), lambda i, off=off: (i + off, 0)),
            pl.BlockSpec((EDGE_DIM, D), lambda i: (0, 0)),
            pl.BlockSpec((1, D), lambda i: (0, 0)),
            pl.BlockSpec((BE, D), lambda i: (i, 0)),
            pl.BlockSpec((D, D), lambda i: (0, 0)),
        ],
        out_specs=pl.BlockSpec((BE, D), lambda i: (i, 0)),
        out_shape=jax.ShapeDtypeStruct((E2, D), jnp.float32),
    )


_edge_upd0 = [_make_edge_upd0(0), _make_edge_upd0(E2 // BE)]


def _node_enc_body(x_ref, w_ref, b_ref, ws_ref, bs_ref, wd_ref,
                   u_ref, p_ref, q_ref):
    u = _dot(x_ref[...], w_ref[...]) + b_ref[...]
    u_ref[...] = u
    p_ref[...] = _dot(u, ws_ref[...]) + bs_ref[...]
    q_ref[...] = _dot(u, wd_ref[...])


_node_enc = pl.pallas_call(
    _node_enc_body,
    grid=(N // BN,),
    in_specs=[
        pl.BlockSpec((BN, NODE_DIM), lambda i: (i, 0)),
        pl.BlockSpec((NODE_DIM, D), lambda i: (0, 0)),
        pl.BlockSpec((1, D), lambda i: (0, 0)),
        pl.BlockSpec((D, D), lambda i: (0, 0)),
        pl.BlockSpec((1, D), lambda i: (0, 0)),
        pl.BlockSpec((D, D), lambda i: (0, 0)),
    ],
    out_specs=[pl.BlockSpec((BN, D), lambda i: (i, 0))] * 3,
    out_shape=[jax.ShapeDtypeStruct((N, D), jnp.float32)] * 3,
)


def _edge_upd_body(u_ref, g_ref, w_ref, o_ref):
    u = u_ref[...]
    pre = _dot(u, w_ref[...]) + g_ref[...]
    o_ref[...] = u + jnp.maximum(pre, 0.0)


_edge_upd = pl.pallas_call(
    _edge_upd_body,
    grid=(E2 // BE,),
    in_specs=[
        pl.BlockSpec((BE, D), lambda i: (i, 0)),
        pl.BlockSpec((BE, D), lambda i: (i, 0)),
        pl.BlockSpec((D, D), lambda i: (0, 0)),
    ],
    out_specs=pl.BlockSpec((BE, D), lambda i: (i, 0)),
    out_shape=jax.ShapeDtypeStruct((E2, D), jnp.float32),
)


def _node_upd_proj_body(u_ref, a0_ref, a1_ref,
                        w1_ref, w2_ref, b_ref,
                        ws_ref, bs_ref, wd_ref, uo_ref, p_ref, q_ref):
    u = u_ref[...]
    agg = a0_ref[...] + a1_ref[...]
    h = _dot(u, w1_ref[...]) + _dot(agg, w2_ref[...]) + b_ref[...]
    un = u + jnp.maximum(h, 0.0)
    uo_ref[...] = un
    p_ref[...] = _dot(un, ws_ref[...]) + bs_ref[...]
    q_ref[...] = _dot(un, wd_ref[...])


_node_upd_proj = pl.pallas_call(
    _node_upd_proj_body,
    grid=(N // BN,),
    in_specs=[pl.BlockSpec((BN, D), lambda i: (i, 0))] * 3 + [
        pl.BlockSpec((D, D), lambda i: (0, 0)),
        pl.BlockSpec((D, D), lambda i: (0, 0)),
        pl.BlockSpec((1, D), lambda i: (0, 0)),
        pl.BlockSpec((D, D), lambda i: (0, 0)),
        pl.BlockSpec((1, D), lambda i: (0, 0)),
        pl.BlockSpec((D, D), lambda i: (0, 0)),
    ],
    out_specs=[pl.BlockSpec((BN, D), lambda i: (i, 0))] * 3,
    out_shape=[jax.ShapeDtypeStruct((N, D), jnp.float32)] * 3,
)


def _node_upd_body(u_ref, a0_ref, a1_ref,
                   w1_ref, w2_ref, b_ref, uo_ref):
    u = u_ref[...]
    agg = a0_ref[...] + a1_ref[...]
    h = _dot(u, w1_ref[...]) + _dot(agg, w2_ref[...]) + b_ref[...]
    uo_ref[...] = u + jnp.maximum(h, 0.0)


_node_upd = pl.pallas_call(
    _node_upd_body,
    grid=(N // BN,),
    in_specs=[pl.BlockSpec((BN, D), lambda i: (i, 0))] * 3 + [
        pl.BlockSpec((D, D), lambda i: (0, 0)),
        pl.BlockSpec((D, D), lambda i: (0, 0)),
        pl.BlockSpec((1, D), lambda i: (0, 0)),
    ],
    out_specs=pl.BlockSpec((BN, D), lambda i: (i, 0)),
    out_shape=jax.ShapeDtypeStruct((N, D), jnp.float32),
)


@functools.partial(
    pl.kernel,
    out_type=jax.ShapeDtypeStruct((E2, D), jnp.float32),
    mesh=_mesh,
    scratch_types=[
        pltpu.VMEM((NIDX,), jnp.int32),
        pltpu.VMEM((NIDX,), jnp.int32),
        pltpu.VMEM((1, REM), jnp.int32),
        pltpu.VMEM((1, REM), jnp.int32),
        pltpu.VMEM((CG,), jnp.int32),
        pltpu.VMEM((CG,), jnp.int32),
        pltpu.VMEM((CG, D), jnp.float32),
        pltpu.VMEM((CG, D), jnp.float32),
        pltpu.VMEM((CG, D), jnp.float32),
        pltpu.VMEM((CG, D), jnp.float32),
        pltpu.VMEM_SHARED((NS * CG, D), jnp.float32),
        pltpu.VMEM_SHARED((NS * CG, D), jnp.float32),
        pltpu.SemaphoreType.DMA,
        pltpu.SemaphoreType.DMA,
        pltpu.SemaphoreType.DMA,
        pltpu.SemaphoreType.DMA,
        pltpu.SemaphoreType.DMA,
    ],
)
def _sc_gather(p_hbm, q_hbm, src_hbm, dst_hbm, srcr_hbm, dstr_hbm, iota_hbm,
               g_hbm,
               sidx, didx, sidxr, didxr, iot, myidx, pA, pB, qA, qB, fA, fB,
               sem_p, sem_q, sem_c, sem_a, sem_w):
    cid = lax.axis_index("c")
    sid = lax.axis_index("s")
    wid = sid * NC + cid
    pltpu.sync_copy(src_hbm.at[wid], sidx)
    pltpu.sync_copy(dst_hbm.at[wid], didx)
    pltpu.sync_copy(srcr_hbm.at[wid], sidxr)
    pltpu.sync_copy(dstr_hbm.at[wid], didxr)
    pltpu.sync_copy(iota_hbm, iot)
    base = sid * CG
    for k in range(CG // 16):
        sl = pl.ds(k * 16, 16)
        myidx[sl] = iot[sl] + base
    ebase = wid * EPW

    def gat2(j, pb, qb):
        hp = pltpu.async_copy(p_hbm.at[sidx.at[pl.ds(j * CG, CG)]], pb, sem_p)
        hq = pltpu.async_copy(q_hbm.at[didx.at[pl.ds(j * CG, CG)]], qb, sem_q)
        return hp, hq

    def cp(pb, fb, sz):
        return pltpu.async_copy(pb.at[pl.ds(0, sz)],
                                fb.at[pl.ds(base, sz)], sem_c)

    def addl(fb, qb, sz):
        return pltpu.async_copy(qb.at[pl.ds(0, sz)],
                                fb.at[myidx.at[pl.ds(0, sz)]], sem_a, add=True)

    def wr(fb, off, sz):
        return pltpu.async_copy(fb.at[pl.ds(base, sz)],
                                g_hbm.at[pl.ds(ebase + off, sz)], sem_w)

    def body(i, carry):
        jA = 2 * i
        jB = jA + 1
        hpA, hqA = gat2(jA, pA, qA)
        hpA.wait()
        cA = cp(pA, fA, CG)
        hqA.wait(); cA.wait()
        aA = addl(fA, qA, CG)
        hpB, hqB = gat2(jB, pB, qB)
        aA.wait()
        wA = wr(fA, jA * CG, CG)
        hpB.wait()
        cB = cp(pB, fB, CG)
        hqB.wait(); cB.wait()
        aB = addl(fB, qB, CG)
        wA.wait(); aB.wait()
        wr(fB, jB * CG, CG).wait()
        return carry

    lax.fori_loop(0, NCG // 2, body, 0)
    hp, hq = gat2(NCG - 1, pA, qA)
    hp.wait()
    cA = cp(pA, fA, CG)
    hq.wait(); cA.wait()
    addl(fA, qA, CG).wait()
    wA = wr(fA, (NCG - 1) * CG, CG)
    hp = pltpu.async_copy(p_hbm.at[sidxr.at[0]], pB.at[pl.ds(0, REM)], sem_p)
    hq = pltpu.async_copy(q_hbm.at[didxr.at[0]], qB.at[pl.ds(0, REM)], sem_q)
    hp.wait()
    cB = cp(pB, fB, REM)
    hq.wait(); cB.wait()
    addl(fB, qB, REM).wait()
    wA.wait()
    wr(fB, NIDX, REM).wait()


@functools.partial(
    pl.kernel,
    out_type=jax.ShapeDtypeStruct((NC, N, D), jnp.float32),
    mesh=_mesh,
    scratch_types=[
        pltpu.VMEM((NIDX,), jnp.int32),
        pltpu.VMEM((1, REM), jnp.int32),
        pltpu.VMEM((CBS, D), jnp.float32),
        pltpu.VMEM((CBS, D), jnp.float32),
        pltpu.VMEM_SHARED((N, D), jnp.float32),
        pltpu.SemaphoreType.DMA,
        pltpu.SemaphoreType.DMA,
    ],
)
def _sc_scatter(uef_hbm, dst_hbm, dstr_hbm, init_hbm, out_hbm,
                didx, didxr, rows0, rows1, acc, sem_l, sem_s):
    cid = lax.axis_index("c")
    sid = lax.axis_index("s")
    wid = sid * NC + cid
    pltpu.sync_copy(init_hbm.at[cid, pl.ds(sid * SN, SN)],
                    acc.at[pl.ds(sid * SN, SN)])

    @pl.when(sid == NS - 1)
    def _zero_rem():
        pltpu.sync_copy(init_hbm.at[cid, pl.ds(NS * SN, SREM)],
                        acc.at[pl.ds(NS * SN, SREM)])

    plsc.subcore_barrier()
    pltpu.sync_copy(dst_hbm.at[wid], didx)
    pltpu.sync_copy(dstr_hbm.at[wid], didxr)
    ebase = wid * EPW

    def start_load(off, sz, buf):
        return pltpu.async_copy(uef_hbm.at[pl.ds(ebase + off, sz)],
                                buf.at[pl.ds(0, sz)], sem_l)

    def start_scat(off, sz, buf):
        return pltpu.async_copy(buf.at[pl.ds(0, sz)],
                                acc.at[didx.at[pl.ds(off, sz)]],
                                sem_s, add=True)

    start_load(0, CBS, rows0).wait()

    def body(i, carry):
        jA = 2 * i + 1
        lA = start_load(jA * CBS, CBS, rows1)
        sP = start_scat((jA - 1) * CBS, CBS, rows0)
        lA.wait(); sP.wait()
        lB = start_load((jA + 1) * CBS, CBS, rows0)
        sA = start_scat(jA * CBS, CBS, rows1)
        lB.wait(); sA.wait()
        return carry

    lax.fori_loop(0, (NCHS - 1) // 2, body, 0)
    lR = pltpu.async_copy(uef_hbm.at[pl.ds(ebase + NIDX, REM)],
                          rows1.at[pl.ds(0, REM)], sem_l)
    sP = start_scat((NCHS - 1) * CBS, CBS, rows0)
    lR.wait(); sP.wait()
    pltpu.async_copy(rows1.at[pl.ds(0, REM)], acc.at[didxr.at[0]],
                     sem_s, add=True).wait()
    plsc.subcore_barrier()
    pltpu.sync_copy(acc.at[pl.ds(sid * SN, SN)],
                    out_hbm.at[cid, pl.ds(sid * SN, SN)])

    @pl.when(sid == NS - 1)
    def _out_rem():
        pltpu.sync_copy(acc.at[pl.ds(NS * SN, SREM)],
                        out_hbm.at[cid, pl.ds(NS * SN, SREM)])


def kernel(nf, ef, edge_index, W_node_enc, b_node_enc, W_edge_enc, b_edge_enc,
           We, be, Wn, bn):
    s2 = edge_index[0].reshape(NH, NW, EPW)
    d2 = edge_index[1].reshape(NH, NW, EPW)
    srcm = s2[..., :NIDX]
    srcr = s2[..., NIDX:].reshape(NH, NW, 1, REM)
    dstm = d2[..., :NIDX]
    dstr = d2[..., NIDX:].reshape(NH, NW, 1, REM)
    zeros_nd = jnp.zeros((NC, N, D), jnp.float32)
    iota = jnp.arange(CG, dtype=jnp.int32)

    unf, P, Q = _node_enc(nf, W_node_enc, b_node_enc.reshape(1, D),
                          We[0, D:2 * D], be[0].reshape(1, D), We[0, 2 * D:])
    uef = [None] * NH

    for l in range(N_LAYER):
        part = zeros_nd
        for h in range(NH):
            g = _sc_gather(P, Q, srcm[h], dstm[h], srcr[h], dstr[h], iota)
            if l == 0:
                uef[h] = _edge_upd0[h](ef, W_edge_enc,
                                       b_edge_enc.reshape(1, D), g, We[0, :D])
            else:
                uef[h] = _edge_upd(uef[h], g, We[l, :D])
            part = _sc_scatter(uef[h], dstm[h], dstr[h], part)
        aggs = (part[0], part[1])
        if l + 1 < N_LAYER:
            unf, P, Q = _node_upd_proj(
                unf, *aggs,
                Wn[l, :D], Wn[l, D:], bn[l].reshape(1, D),
                We[l + 1, D:2 * D], be[l + 1].reshape(1, D), We[l + 1, 2 * D:])
        else:
            unf = _node_upd(unf, *aggs,
                            Wn[l, :D], Wn[l, D:], bn[l].reshape(1, D))
    return unf, jnp.concatenate(uef, axis=0)
